# Initial kernel scaffold; baseline (speedup 1.0000x reference)
import jax
import jax.numpy as jnp
from jax.experimental import pallas as pl


def kernel(x, edge_index_0, edge_index_1, edge_index_2, W0_0, b0_0, W0_1, b0_1, W0_2, b0_2, W1_0, b1_0, W1_1, b1_1, W1_2, b1_2, W2_0, b2_0, W2_1, b2_1, W2_2, b2_2):
    raise NotImplementedError("write your pallas kernel here")


# speedup vs baseline: 4.4089x; 4.4089x over previous
import functools

import jax
import jax.numpy as jnp
from jax import lax
from jax.experimental import pallas as pl
from jax.experimental.pallas import tpu as pltpu
from jax.experimental.pallas import tpu_sc as plsc

N = 10000
E = 100000
IN_F = 128
HID = 256
NCLS = 40
NET = 3

NC = 2
NS = 16
NW = NC * NS
CHUNK = 128
N_PAD = 10240
PAD_IDX = N
CPW = 25
E_PAD = NW * CPW * CHUNK
STRIPE = N_PAD // NS
WB = STRIPE // CHUNK


def _make_sc_agg(n_passes, feat, etype_of_pass):
    mesh = plsc.VectorSubcoreMesh(
        core_axis_name="c", subcore_axis_name="s", num_cores=NC,
        num_subcores=NS)

    def body(*refs):
        tabs = refs[:n_passes]
        src_r, dst_r, zrow_r = refs[n_passes:n_passes + 3]
        out_r = refs[n_passes + 3]
        rows0, rows1, sidx, didx, acc, sem0, sem1 = refs[n_passes + 4:]

        c = lax.axis_index("c")
        s = lax.axis_index("s")
        w = c * NS + s
        rb = s * STRIPE

        for p in range(n_passes):
            e = etype_of_pass[p]
            pltpu.sync_copy(zrow_r, rows0)
            for j in range(WB):
                pltpu.sync_copy(rows0, acc.at[pl.ds(rb + j * CHUNK, CHUNK)])
            pltpu.sync_copy(src_r.at[e, w], sidx)
            pltpu.sync_copy(dst_r.at[e, w], didx)
            plsc.subcore_barrier()

            def step(i, carry):
                j = 2 * i
                d0 = pltpu.async_copy(tabs[p].at[sidx.at[j]], rows0, sem0)
                d1 = pltpu.async_copy(tabs[p].at[sidx.at[j + 1]], rows1, sem1)
                d0.wait()
                pltpu.sync_copy(rows0, acc.at[didx.at[j]], add=True)
                d1.wait()
                pltpu.sync_copy(rows1, acc.at[didx.at[j + 1]], add=True)
                return carry

            lax.fori_loop(0, CPW // 2, step, 0)
            d0 = pltpu.async_copy(tabs[p].at[sidx.at[CPW - 1]], rows0, sem0)
            d0.wait()
            pltpu.sync_copy(rows0, acc.at[didx.at[CPW - 1]], add=True)
            plsc.subcore_barrier()

            for j in range(WB):
                r = rb + j * CHUNK
                pltpu.sync_copy(acc.at[pl.ds(r, CHUNK)], rows0)
                pltpu.sync_copy(rows0, out_r.at[p, c, pl.ds(r, CHUNK)])
            plsc.subcore_barrier()

    tab_types = [jax.ShapeDtypeStruct((N_PAD, feat), jnp.float32)
                 for _ in range(n_passes)]

    def run(tables, src3, dst3, zrow):
        f = pl.kernel(
            body,
            out_type=jax.ShapeDtypeStruct((n_passes, NC, N_PAD, feat),
                                          jnp.float32),
            mesh=mesh,
            scratch_types=[
                pltpu.VMEM((CHUNK, feat), jnp.float32),
                pltpu.VMEM((CHUNK, feat), jnp.float32),
                pltpu.VMEM((CPW, CHUNK), jnp.int32),
                pltpu.VMEM((CPW, CHUNK), jnp.int32),
                pltpu.VMEM_SHARED((N_PAD, feat), jnp.float32),
                pltpu.SemaphoreType.DMA,
                pltpu.SemaphoreType.DMA,
            ],
        )
        return f(*tables, src3, dst3, zrow)

    del tab_types
    return run


def _make_sc_degrees():
    mesh = plsc.VectorSubcoreMesh(
        core_axis_name="c", subcore_axis_name="s", num_cores=NC,
        num_subcores=NS)

    def body(src_r, dst_r, ones_r, zrow_r, out_r,
             ones_v, stg, didx, acc, sem0):
        del sem0
        c = lax.axis_index("c")
        s = lax.axis_index("s")
        w = c * NS + s
        rb = s * STRIPE

        pltpu.sync_copy(ones_r, ones_v)

        for p in range(6):
            e = p // 2
            idx_r = src_r if p % 2 == 0 else dst_r
            pltpu.sync_copy(zrow_r, stg)
            for j in range(WB):
                pltpu.sync_copy(stg, acc.at[pl.ds(rb + j * CHUNK, CHUNK)])
            pltpu.sync_copy(idx_r.at[e, w], didx)
            plsc.subcore_barrier()

            def step(j, carry):
                pltpu.sync_copy(ones_v, acc.at[didx.at[j]], add=True)
                return carry

            lax.fori_loop(0, CPW, step, 0)
            plsc.subcore_barrier()

            for j in range(WB):
                r = rb + j * CHUNK
                pltpu.sync_copy(acc.at[pl.ds(r, CHUNK)], stg)
                pltpu.sync_copy(stg, out_r.at[p, c, pl.ds(r, CHUNK)])
            plsc.subcore_barrier()

    def run(src3, dst3, ones_row, zrow):
        f = pl.kernel(
            body,
            out_type=jax.ShapeDtypeStruct((6, NC, N_PAD, IN_F), jnp.float32),
            mesh=mesh,
            scratch_types=[
                pltpu.VMEM((CHUNK, IN_F), jnp.float32),
                pltpu.VMEM((CHUNK, IN_F), jnp.float32),
                pltpu.VMEM((CPW, CHUNK), jnp.int32),
                pltpu.VMEM_SHARED((N_PAD, IN_F), jnp.float32),
                pltpu.SemaphoreType.DMA,
            ],
        )
        return f(src3, dst3, ones_row, zrow)

    return run


def _norm_col(degp_pair):
    deg = degp_pair[0, :, 0:1] + degp_pair[1, :, 0:1]
    return jnp.where(deg > 0.0, lax.rsqrt(jnp.maximum(deg, 1.0)), 0.0)


def _k0_body(degp_r, x_r, xs0_r, xs1_r, xs2_r):
    degp = degp_r[...]
    x = x_r[...]
    outs = (xs0_r, xs1_r, xs2_r)
    for e in range(NET):
        ns = _norm_col(degp[2 * e])
        outs[e][...] = x * ns


def _k1_body(degp_r, a0_r, w_r, b_r, *y_refs):
    degp = degp_r[...]
    a0 = a0_r[...]
    w = w_r[...]
    b = b_r[...]
    bbar = (b[0] + b[1] + b[2]) * (1.0 / 3.0)
    acc = None
    for e in range(NET):
        nd = _norm_col(degp[2 * e + 1])
        part = (a0[e, 0] + a0[e, 1]) * nd
        t = jnp.dot(part, w[e], preferred_element_type=jnp.float32)
        acc = t if acc is None else acc + t
    h1 = jnp.maximum(acc * (1.0 / 3.0) + bbar[None, :], 0.0)
    for e in range(NET):
        ns = _norm_col(degp[2 * e])
        t = h1 * ns
        y_refs[2 * e][...] = t[:, :IN_F]
        y_refs[2 * e + 1][...] = t[:, IN_F:]


def _k2_body(degp_r, a1_r, w1_r, b1_r, w2_r, z0_r, z1_r, z2_r):
    degp = degp_r[...]
    a1 = a1_r[...]
    w1 = w1_r[...]
    b1 = b1_r[...]
    w2 = w2_r[...]
    bbar = (b1[0] + b1[1] + b1[2]) * (1.0 / 3.0)
    acc = None
    for e in range(NET):
        nd = _norm_col(degp[2 * e + 1])
        t = None
        for h in range(2):
            part = a1[2 * e + h, 0] + a1[2 * e + h, 1]
            m = jnp.dot(part, w1[e, h * IN_F:(h + 1) * IN_F, :],
                        preferred_element_type=jnp.float32)
            t = m if t is None else t + m
        t = t * nd
        acc = t if acc is None else acc + t
    h2 = jnp.maximum(acc * (1.0 / 3.0) + bbar[None, :], 0.0)
    z_refs = (z0_r, z1_r, z2_r)
    for e in range(NET):
        ns = _norm_col(degp[2 * e])
        z_refs[e][...] = jnp.dot(h2 * ns, w2[e],
                                 preferred_element_type=jnp.float32)


def _k3_body(degp_r, a2_r, b2_r, logits_r, probs_r):
    degp = degp_r[...]
    a2 = a2_r[...]
    b2 = b2_r[...]
    bbar = (b2[0] + b2[1] + b2[2]) * (1.0 / 3.0)
    acc = None
    for e in range(NET):
        nd = _norm_col(degp[2 * e + 1])
        t = (a2[e, 0] + a2[e, 1]) * nd
        acc = t if acc is None else acc + t
    logits = acc * (1.0 / 3.0) + bbar[None, :]
    logits_r[...] = logits
    mask = lax.broadcasted_iota(jnp.int32, logits.shape, 1) < NCLS
    xm = jnp.where(mask, logits, -1e30)
    mx = jnp.max(xm, axis=1, keepdims=True)
    ex = jnp.where(mask, jnp.exp(xm - mx), 0.0)
    probs_r[...] = ex / jnp.sum(ex, axis=1, keepdims=True)


def _row_spec(rank_prefix, r, minor):
    def imap(i):
        return tuple([0] * len(rank_prefix) + [i, 0])
    return pl.BlockSpec(tuple(rank_prefix) + (r, minor), imap)


def _full_spec(shape):
    return pl.BlockSpec(shape, lambda i: tuple(0 for _ in shape))


def kernel(x, edge_index_0, edge_index_1, edge_index_2,
           W0_0, b0_0, W0_1, b0_1, W0_2, b0_2,
           W1_0, b1_0, W1_1, b1_1, W1_2, b1_2,
           W2_0, b2_0, W2_1, b2_1, W2_2, b2_2):
    f32 = jnp.float32

    x_pad = jnp.pad(x, ((0, N_PAD - N), (0, 0)))
    pad_e = PAD_IDX + jnp.arange(E_PAD - E, dtype=jnp.int32) % (N_PAD - N)
    srcs, dsts = [], []
    for ei in (edge_index_0, edge_index_1, edge_index_2):
        srcs.append(jnp.concatenate([ei[0], pad_e]).reshape(NW, CPW, CHUNK))
        dsts.append(jnp.concatenate([ei[1], pad_e]).reshape(NW, CPW, CHUNK))
    src3 = jnp.stack(srcs)
    dst3 = jnp.stack(dsts)

    W0s = jnp.stack([W0_0, W0_1, W0_2])
    b0s = jnp.stack([b0_0, b0_1, b0_2])
    W1s = jnp.stack([W1_0, W1_1, W1_2])
    b1s = jnp.stack([b1_0, b1_1, b1_2])
    W2s = jnp.pad(jnp.stack([W2_0, W2_1, W2_2]), ((0, 0), (0, 0), (0, 88)))
    b2s = jnp.pad(jnp.stack([b2_0, b2_1, b2_2]), ((0, 0), (0, 88)))

    zrow128 = jnp.zeros((CHUNK, IN_F), f32)
    ones128 = jnp.ones((CHUNK, IN_F), f32)

    degp = _make_sc_degrees()(src3, dst3, ones128, zrow128)

    r0 = 256
    grid0 = (N_PAD // r0,)
    xs = pl.pallas_call(
        _k0_body,
        grid=grid0,
        in_specs=[_row_spec((6, NC), r0, IN_F), _row_spec((), r0, IN_F)],
        out_specs=[_row_spec((), r0, IN_F)] * 3,
        out_shape=[jax.ShapeDtypeStruct((N_PAD, IN_F), f32)] * 3,
    )(degp, x_pad)

    agg0 = _make_sc_agg(3, IN_F, [0, 1, 2])(xs, src3, dst3, zrow128)

    r1 = 256
    ys = pl.pallas_call(
        _k1_body,
        grid=(N_PAD // r1,),
        in_specs=[
            _row_spec((6, NC), r1, IN_F),
            _row_spec((3, NC), r1, IN_F),
            _full_spec((3, IN_F, HID)),
            _full_spec((3, HID)),
        ],
        out_specs=[_row_spec((), r1, IN_F)] * 6,
        out_shape=[jax.ShapeDtypeStruct((N_PAD, IN_F), f32)] * 6,
    )(degp, agg0, W0s, b0s)

    agg1 = _make_sc_agg(6, IN_F, [0, 0, 1, 1, 2, 2])(ys, src3, dst3, zrow128)

    zs = pl.pallas_call(
        _k2_body,
        grid=(N_PAD // r1,),
        in_specs=[
            _row_spec((6, NC), r1, IN_F),
            _row_spec((6, NC), r1, IN_F),
            _full_spec((3, HID, HID)),
            _full_spec((3, HID)),
            _full_spec((3, HID, IN_F)),
        ],
        out_specs=[_row_spec((), r1, IN_F)] * 3,
        out_shape=[jax.ShapeDtypeStruct((N_PAD, IN_F), f32)] * 3,
    )(degp, agg1, W1s, b1s, W2s)

    agg2 = _make_sc_agg(3, IN_F, [0, 1, 2])(zs, src3, dst3, zrow128)

    logits_pad, probs_pad = pl.pallas_call(
        _k3_body,
        grid=(N_PAD // r1,),
        in_specs=[
            _row_spec((6, NC), r1, IN_F),
            _row_spec((3, NC), r1, IN_F),
            _full_spec((3, IN_F)),
        ],
        out_specs=[_row_spec((), r1, IN_F)] * 2,
        out_shape=[jax.ShapeDtypeStruct((N_PAD, IN_F), f32)] * 2,
    )(degp, agg2, b2s)

    return logits_pad[:N, :NCLS], probs_pad[:N, :NCLS]


# speedup vs baseline: 5.1670x; 1.1720x over previous
import functools

import jax
import jax.numpy as jnp
from jax import lax
from jax.experimental import pallas as pl
from jax.experimental.pallas import tpu as pltpu
from jax.experimental.pallas import tpu_sc as plsc

N = 10000
E = 100000
IN_F = 128
HID = 256
NCLS = 40
NET = 3

NC = 2
NS = 16
NW = NC * NS
CHUNK = 128
N_PAD = 10240
PAD_IDX = N
CPW = 25
E_PAD = NW * CPW * CHUNK
STRIPE = N_PAD // NS
WB = STRIPE // CHUNK


def _make_sc_agg(n_passes, feat, etype_of_pass):
    mesh = plsc.VectorSubcoreMesh(
        core_axis_name="c", subcore_axis_name="s", num_cores=NC,
        num_subcores=NS)

    def body(*refs):
        tabs = refs[:n_passes]
        src_r, dst_r, zrow_r = refs[n_passes:n_passes + 3]
        out_r = refs[n_passes + 3]
        rows0, rows1, sidx, didx, acc, sem0, sem1 = refs[n_passes + 4:]

        c = lax.axis_index("c")
        s = lax.axis_index("s")
        w = c * NS + s
        rb = s * STRIPE

        for p in range(n_passes):
            e = etype_of_pass[p]
            pltpu.sync_copy(zrow_r, rows0)
            for j in range(WB):
                pltpu.sync_copy(rows0, acc.at[pl.ds(rb + j * CHUNK, CHUNK)])
            pltpu.sync_copy(src_r.at[e, w], sidx)
            pltpu.sync_copy(dst_r.at[e, w], didx)
            plsc.subcore_barrier()

            pltpu.async_copy(tabs[p].at[sidx.at[0]], rows0, sem0)

            def step(i, carry):
                j = 2 * i
                pltpu.async_copy(tabs[p].at[sidx.at[j + 1]], rows1, sem1)
                pltpu.make_async_copy(
                    tabs[p].at[sidx.at[j]], rows0, sem0).wait()
                pltpu.sync_copy(rows0, acc.at[didx.at[j]], add=True)
                pltpu.async_copy(tabs[p].at[sidx.at[j + 2]], rows0, sem0)
                pltpu.make_async_copy(
                    tabs[p].at[sidx.at[j + 1]], rows1, sem1).wait()
                pltpu.sync_copy(rows1, acc.at[didx.at[j + 1]], add=True)
                return carry

            lax.fori_loop(0, (CPW - 1) // 2, step, 0)
            pltpu.make_async_copy(
                tabs[p].at[sidx.at[CPW - 1]], rows0, sem0).wait()
            pltpu.sync_copy(rows0, acc.at[didx.at[CPW - 1]], add=True)
            plsc.subcore_barrier()

            pltpu.sync_copy(acc.at[pl.ds(rb, STRIPE)],
                            out_r.at[p, c, pl.ds(rb, STRIPE)])
            plsc.subcore_barrier()

    tab_types = [jax.ShapeDtypeStruct((N_PAD, feat), jnp.float32)
                 for _ in range(n_passes)]

    def run(tables, src3, dst3, zrow):
        f = pl.kernel(
            body,
            out_type=jax.ShapeDtypeStruct((n_passes, NC, N_PAD, feat),
                                          jnp.float32),
            mesh=mesh,
            scratch_types=[
                pltpu.VMEM((CHUNK, feat), jnp.float32),
                pltpu.VMEM((CHUNK, feat), jnp.float32),
                pltpu.VMEM((CPW, CHUNK), jnp.int32),
                pltpu.VMEM((CPW, CHUNK), jnp.int32),
                pltpu.VMEM_SHARED((N_PAD, feat), jnp.float32),
                pltpu.SemaphoreType.DMA,
                pltpu.SemaphoreType.DMA,
            ],
        )
        return f(*tables, src3, dst3, zrow)

    del tab_types
    return run


def _make_sc_degrees():
    mesh = plsc.VectorSubcoreMesh(
        core_axis_name="c", subcore_axis_name="s", num_cores=NC,
        num_subcores=NS)

    def body(src_r, dst_r, ones_r, zrow_r, out_r,
             ones_v, stg, didx, acc, sem0):
        c = lax.axis_index("c")
        s = lax.axis_index("s")
        w = c * NS + s
        rb = s * STRIPE

        pltpu.sync_copy(ones_r, ones_v)

        for p in range(6):
            e = p // 2
            idx_r = src_r if p % 2 == 0 else dst_r
            pltpu.sync_copy(zrow_r, stg)
            for j in range(WB):
                pltpu.sync_copy(stg, acc.at[pl.ds(rb + j * CHUNK, CHUNK)])
            pltpu.sync_copy(idx_r.at[e, w], didx)
            plsc.subcore_barrier()

            def fire(j, carry):
                pltpu.async_copy(ones_v, acc.at[didx.at[j]], sem0, add=True)
                return carry

            lax.fori_loop(0, CPW, fire, 0)

            def drain(j, carry):
                pltpu.make_async_copy(ones_v, acc.at[didx.at[0]], sem0).wait()
                return carry

            lax.fori_loop(0, CPW, drain, 0)
            plsc.subcore_barrier()

            pltpu.sync_copy(acc.at[pl.ds(rb, STRIPE)],
                            out_r.at[p, c, pl.ds(rb, STRIPE)])
            plsc.subcore_barrier()

    def run(src3, dst3, ones_row, zrow):
        f = pl.kernel(
            body,
            out_type=jax.ShapeDtypeStruct((6, NC, N_PAD, IN_F), jnp.float32),
            mesh=mesh,
            scratch_types=[
                pltpu.VMEM((CHUNK, IN_F), jnp.float32),
                pltpu.VMEM((CHUNK, IN_F), jnp.float32),
                pltpu.VMEM((CPW, CHUNK), jnp.int32),
                pltpu.VMEM_SHARED((N_PAD, IN_F), jnp.float32),
                pltpu.SemaphoreType.DMA,
            ],
        )
        return f(src3, dst3, ones_row, zrow)

    return run


def _norm_col(degp_pair):
    deg = degp_pair[0, :, 0:1] + degp_pair[1, :, 0:1]
    return jnp.where(deg > 0.0, lax.rsqrt(jnp.maximum(deg, 1.0)), 0.0)


def _k0_body(degp_r, x_r, xs0_r, xs1_r, xs2_r):
    degp = degp_r[...]
    x = x_r[...]
    outs = (xs0_r, xs1_r, xs2_r)
    for e in range(NET):
        ns = _norm_col(degp[2 * e])
        outs[e][...] = x * ns


def _k1_body(degp_r, a0_r, w_r, b_r, *y_refs):
    degp = degp_r[...]
    a0 = a0_r[...]
    w = w_r[...]
    b = b_r[...]
    bbar = (b[0] + b[1] + b[2]) * (1.0 / 3.0)
    acc = None
    for e in range(NET):
        nd = _norm_col(degp[2 * e + 1])
        part = (a0[e, 0] + a0[e, 1]) * nd
        t = jnp.dot(part, w[e], preferred_element_type=jnp.float32)
        acc = t if acc is None else acc + t
    h1 = jnp.maximum(acc * (1.0 / 3.0) + bbar[None, :], 0.0)
    for e in range(NET):
        ns = _norm_col(degp[2 * e])
        t = h1 * ns
        y_refs[2 * e][...] = t[:, :IN_F]
        y_refs[2 * e + 1][...] = t[:, IN_F:]


def _k2_body(degp_r, a1_r, w1_r, b1_r, w2_r, z0_r, z1_r, z2_r):
    degp = degp_r[...]
    a1 = a1_r[...]
    w1 = w1_r[...]
    b1 = b1_r[...]
    w2 = w2_r[...]
    bbar = (b1[0] + b1[1] + b1[2]) * (1.0 / 3.0)
    acc = None
    for e in range(NET):
        nd = _norm_col(degp[2 * e + 1])
        t = None
        for h in range(2):
            part = a1[2 * e + h, 0] + a1[2 * e + h, 1]
            m = jnp.dot(part, w1[e, h * IN_F:(h + 1) * IN_F, :],
                        preferred_element_type=jnp.float32)
            t = m if t is None else t + m
        t = t * nd
        acc = t if acc is None else acc + t
    h2 = jnp.maximum(acc * (1.0 / 3.0) + bbar[None, :], 0.0)
    z_refs = (z0_r, z1_r, z2_r)
    for e in range(NET):
        ns = _norm_col(degp[2 * e])
        z_refs[e][...] = jnp.dot(h2 * ns, w2[e],
                                 preferred_element_type=jnp.float32)


def _k3_body(degp_r, a2_r, b2_r, logits_r, probs_r):
    degp = degp_r[...]
    a2 = a2_r[...]
    b2 = b2_r[...]
    bbar = (b2[0] + b2[1] + b2[2]) * (1.0 / 3.0)
    acc = None
    for e in range(NET):
        nd = _norm_col(degp[2 * e + 1])
        t = (a2[e, 0] + a2[e, 1]) * nd
        acc = t if acc is None else acc + t
    logits = acc * (1.0 / 3.0) + bbar[None, :]
    logits_r[...] = logits
    mask = lax.broadcasted_iota(jnp.int32, logits.shape, 1) < NCLS
    xm = jnp.where(mask, logits, -1e30)
    mx = jnp.max(xm, axis=1, keepdims=True)
    ex = jnp.where(mask, jnp.exp(xm - mx), 0.0)
    probs_r[...] = ex / jnp.sum(ex, axis=1, keepdims=True)


def _row_spec(rank_prefix, r, minor):
    def imap(i):
        return tuple([0] * len(rank_prefix) + [i, 0])
    return pl.BlockSpec(tuple(rank_prefix) + (r, minor), imap)


def _full_spec(shape):
    return pl.BlockSpec(shape, lambda i: tuple(0 for _ in shape))


def kernel(x, edge_index_0, edge_index_1, edge_index_2,
           W0_0, b0_0, W0_1, b0_1, W0_2, b0_2,
           W1_0, b1_0, W1_1, b1_1, W1_2, b1_2,
           W2_0, b2_0, W2_1, b2_1, W2_2, b2_2):
    f32 = jnp.float32

    x_pad = jnp.pad(x, ((0, N_PAD - N), (0, 0)))
    pad_e = PAD_IDX + jnp.arange(E_PAD - E, dtype=jnp.int32) % (N_PAD - N)
    srcs, dsts = [], []
    for ei in (edge_index_0, edge_index_1, edge_index_2):
        srcs.append(jnp.concatenate([ei[0], pad_e]).reshape(NW, CPW, CHUNK))
        dsts.append(jnp.concatenate([ei[1], pad_e]).reshape(NW, CPW, CHUNK))
    src3 = jnp.stack(srcs)
    dst3 = jnp.stack(dsts)

    W0s = jnp.stack([W0_0, W0_1, W0_2])
    b0s = jnp.stack([b0_0, b0_1, b0_2])
    W1s = jnp.stack([W1_0, W1_1, W1_2])
    b1s = jnp.stack([b1_0, b1_1, b1_2])
    W2s = jnp.pad(jnp.stack([W2_0, W2_1, W2_2]), ((0, 0), (0, 0), (0, 88)))
    b2s = jnp.pad(jnp.stack([b2_0, b2_1, b2_2]), ((0, 0), (0, 88)))

    zrow128 = jnp.zeros((CHUNK, IN_F), f32)
    ones128 = jnp.ones((CHUNK, IN_F), f32)

    degp = _make_sc_degrees()(src3, dst3, ones128, zrow128)

    r0 = 256
    grid0 = (N_PAD // r0,)
    xs = pl.pallas_call(
        _k0_body,
        grid=grid0,
        in_specs=[_row_spec((6, NC), r0, IN_F), _row_spec((), r0, IN_F)],
        out_specs=[_row_spec((), r0, IN_F)] * 3,
        out_shape=[jax.ShapeDtypeStruct((N_PAD, IN_F), f32)] * 3,
    )(degp, x_pad)

    agg0 = _make_sc_agg(3, IN_F, [0, 1, 2])(xs, src3, dst3, zrow128)

    r1 = 256
    ys = pl.pallas_call(
        _k1_body,
        grid=(N_PAD // r1,),
        in_specs=[
            _row_spec((6, NC), r1, IN_F),
            _row_spec((3, NC), r1, IN_F),
            _full_spec((3, IN_F, HID)),
            _full_spec((3, HID)),
        ],
        out_specs=[_row_spec((), r1, IN_F)] * 6,
        out_shape=[jax.ShapeDtypeStruct((N_PAD, IN_F), f32)] * 6,
    )(degp, agg0, W0s, b0s)

    agg1 = _make_sc_agg(6, IN_F, [0, 0, 1, 1, 2, 2])(ys, src3, dst3, zrow128)

    zs = pl.pallas_call(
        _k2_body,
        grid=(N_PAD // r1,),
        in_specs=[
            _row_spec((6, NC), r1, IN_F),
            _row_spec((6, NC), r1, IN_F),
            _full_spec((3, HID, HID)),
            _full_spec((3, HID)),
            _full_spec((3, HID, IN_F)),
        ],
        out_specs=[_row_spec((), r1, IN_F)] * 3,
        out_shape=[jax.ShapeDtypeStruct((N_PAD, IN_F), f32)] * 3,
    )(degp, agg1, W1s, b1s, W2s)

    agg2 = _make_sc_agg(3, IN_F, [0, 1, 2])(zs, src3, dst3, zrow128)

    logits_pad, probs_pad = pl.pallas_call(
        _k3_body,
        grid=(N_PAD // r1,),
        in_specs=[
            _row_spec((6, NC), r1, IN_F),
            _row_spec((3, NC), r1, IN_F),
            _full_spec((3, IN_F)),
        ],
        out_specs=[_row_spec((), r1, IN_F)] * 2,
        out_shape=[jax.ShapeDtypeStruct((N_PAD, IN_F), f32)] * 2,
    )(degp, agg2, b2s)

    return logits_pad[:N, :NCLS], probs_pad[:N, :NCLS]


# speedup vs baseline: 6.0770x; 1.1761x over previous
import functools

import jax
import jax.numpy as jnp
from jax import lax
from jax.experimental import pallas as pl
from jax.experimental.pallas import tpu as pltpu
from jax.experimental.pallas import tpu_sc as plsc

N = 10000
E = 100000
IN_F = 128
HID = 256
NCLS = 40
NET = 3

NC = 2
NS = 16
NW = NC * NS
CHUNK = 128
N_PAD = 10240
PAD_IDX = N
CPW = 25
E_PAD = NW * CPW * CHUNK
STRIPE = N_PAD // NS
WB = STRIPE // CHUNK


def _make_sc_agg(n_passes, feat, etype_of_pass):
    mesh = plsc.VectorSubcoreMesh(
        core_axis_name="c", subcore_axis_name="s", num_cores=NC,
        num_subcores=NS)

    def body(*refs):
        tabs = refs[:n_passes]
        src_r, dst_r, zrow_r = refs[n_passes:n_passes + 3]
        out_r = refs[n_passes + 3]
        rows0, rows1, sidx, didx, acc, sem0, sem1 = refs[n_passes + 4:]

        c = lax.axis_index("c")
        s = lax.axis_index("s")
        w = c * NS + s
        rb = s * STRIPE

        for p in range(n_passes):
            e = etype_of_pass[p]
            pltpu.sync_copy(zrow_r, rows0)
            for j in range(WB):
                pltpu.sync_copy(rows0, acc.at[pl.ds(rb + j * CHUNK, CHUNK)])
            pltpu.sync_copy(src_r.at[e, w], sidx)
            pltpu.sync_copy(dst_r.at[e, w], didx)
            plsc.subcore_barrier()

            pltpu.async_copy(tabs[p].at[sidx.at[0]], rows0, sem0)

            def step(i, carry):
                j = 2 * i
                pltpu.async_copy(tabs[p].at[sidx.at[j + 1]], rows1, sem1)
                pltpu.make_async_copy(
                    tabs[p].at[sidx.at[j]], rows0, sem0).wait()
                pltpu.sync_copy(rows0, acc.at[didx.at[j]], add=True)
                pltpu.async_copy(tabs[p].at[sidx.at[j + 2]], rows0, sem0)
                pltpu.make_async_copy(
                    tabs[p].at[sidx.at[j + 1]], rows1, sem1).wait()
                pltpu.sync_copy(rows1, acc.at[didx.at[j + 1]], add=True)
                return carry

            lax.fori_loop(0, (CPW - 1) // 2, step, 0)
            pltpu.make_async_copy(
                tabs[p].at[sidx.at[CPW - 1]], rows0, sem0).wait()
            pltpu.sync_copy(rows0, acc.at[didx.at[CPW - 1]], add=True)
            plsc.subcore_barrier()

            pltpu.sync_copy(acc.at[pl.ds(rb, STRIPE)],
                            out_r.at[p, c, pl.ds(rb, STRIPE)])
            plsc.subcore_barrier()

    tab_types = [jax.ShapeDtypeStruct((N_PAD, feat), jnp.float32)
                 for _ in range(n_passes)]

    def run(tables, src3, dst3, zrow):
        f = pl.kernel(
            body,
            out_type=jax.ShapeDtypeStruct((n_passes, NC, N_PAD, feat),
                                          jnp.float32),
            mesh=mesh,
            scratch_types=[
                pltpu.VMEM((CHUNK, feat), jnp.float32),
                pltpu.VMEM((CHUNK, feat), jnp.float32),
                pltpu.VMEM((CPW, CHUNK), jnp.int32),
                pltpu.VMEM((CPW, CHUNK), jnp.int32),
                pltpu.VMEM_SHARED((N_PAD, feat), jnp.float32),
                pltpu.SemaphoreType.DMA,
                pltpu.SemaphoreType.DMA,
            ],
        )
        return f(*tables, src3, dst3, zrow)

    del tab_types
    return run


def _make_sc_degrees():
    mesh = plsc.VectorSubcoreMesh(
        core_axis_name="c", subcore_axis_name="s", num_cores=NC,
        num_subcores=NS)
    DR = N_PAD // IN_F

    def body(src_r, dst_r, zrow_r, rowidx_r, out_r,
             deg2d, didx, rowidx, acc, sem0):
        del sem0
        c = lax.axis_index("c")
        s = lax.axis_index("s")
        w = c * NS + s

        pltpu.sync_copy(rowidx_r, rowidx)
        ones16 = jnp.ones((16,), jnp.float32)

        for p in range(6):
            e = p // 2
            idx_r = src_r if p % 2 == 0 else dst_r
            pltpu.sync_copy(zrow_r.at[pl.ds(0, 8)], acc.at[pl.ds(s * 8, 8)])
            pltpu.sync_copy(zrow_r.at[pl.ds(0, DR)], deg2d)
            pltpu.sync_copy(idx_r.at[e, w], didx)

            def step(g, carry):
                r = g // 8
                col = (g % 8) * 16
                v = didx[r, pl.ds(col, 16)]
                plsc.addupdate_scatter(
                    deg2d,
                    [lax.shift_right_logical(v, 7),
                     lax.bitwise_and(v, 127)],
                    ones16)
                return carry

            lax.fori_loop(0, CPW * 8, step, 0)
            plsc.subcore_barrier()
            pltpu.sync_copy(deg2d, acc.at[rowidx], add=True)
            plsc.subcore_barrier()
            pltpu.sync_copy(acc.at[pl.ds(s * 8, 8)],
                            out_r.at[p, c, pl.ds(s * 8, 8)])
            plsc.subcore_barrier()

    def run(src3, dst3, zrow, rowidx):
        f = pl.kernel(
            body,
            out_type=jax.ShapeDtypeStruct((6, NC, IN_F, IN_F), jnp.float32),
            mesh=mesh,
            compiler_params=pltpu.CompilerParams(needs_layout_passes=False),
            scratch_types=[
                pltpu.VMEM((DR, IN_F), jnp.float32),
                pltpu.VMEM((CPW, CHUNK), jnp.int32),
                pltpu.VMEM((DR,), jnp.int32),
                pltpu.VMEM_SHARED((IN_F, IN_F), jnp.float32),
                pltpu.SemaphoreType.DMA,
            ],
        )
        return f(src3, dst3, zrow, rowidx)

    return run


def _norm_col(deg):
    return jnp.where(deg > 0.0, lax.rsqrt(jnp.maximum(deg, 1.0)), 0.0)


def _k0_body(degp_r, x_r, xs0_r, xs1_r, xs2_r):
    degp = degp_r[...]
    x = x_r[...]
    outs = (xs0_r, xs1_r, xs2_r)
    for e in range(NET):
        ns = _norm_col(degp[2 * e])
        outs[e][...] = x * ns


def _k1_body(degp_r, a0_r, w_r, b_r, *y_refs):
    degp = degp_r[...]
    a0 = a0_r[...]
    w = w_r[...]
    b = b_r[...]
    bbar = (b[0] + b[1] + b[2]) * (1.0 / 3.0)
    acc = None
    for e in range(NET):
        nd = _norm_col(degp[2 * e + 1])
        part = (a0[e, 0] + a0[e, 1]) * nd
        t = jnp.dot(part, w[e], preferred_element_type=jnp.float32)
        acc = t if acc is None else acc + t
    h1 = jnp.maximum(acc * (1.0 / 3.0) + bbar[None, :], 0.0)
    for e in range(NET):
        ns = _norm_col(degp[2 * e])
        t = h1 * ns
        y_refs[2 * e][...] = t[:, :IN_F]
        y_refs[2 * e + 1][...] = t[:, IN_F:]


def _k2_body(degp_r, a1_r, w1_r, b1_r, w2_r, z0_r, z1_r, z2_r):
    degp = degp_r[...]
    a1 = a1_r[...]
    w1 = w1_r[...]
    b1 = b1_r[...]
    w2 = w2_r[...]
    bbar = (b1[0] + b1[1] + b1[2]) * (1.0 / 3.0)
    acc = None
    for e in range(NET):
        nd = _norm_col(degp[2 * e + 1])
        t = None
        for h in range(2):
            part = a1[2 * e + h, 0] + a1[2 * e + h, 1]
            m = jnp.dot(part, w1[e, h * IN_F:(h + 1) * IN_F, :],
                        preferred_element_type=jnp.float32)
            t = m if t is None else t + m
        t = t * nd
        acc = t if acc is None else acc + t
    h2 = jnp.maximum(acc * (1.0 / 3.0) + bbar[None, :], 0.0)
    z_refs = (z0_r, z1_r, z2_r)
    for e in range(NET):
        ns = _norm_col(degp[2 * e])
        z_refs[e][...] = jnp.dot(h2 * ns, w2[e],
                                 preferred_element_type=jnp.float32)


def _k3_body(degp_r, a2_r, b2_r, logits_r, probs_r):
    degp = degp_r[...]
    a2 = a2_r[...]
    b2 = b2_r[...]
    bbar = (b2[0] + b2[1] + b2[2]) * (1.0 / 3.0)
    acc = None
    for e in range(NET):
        nd = _norm_col(degp[2 * e + 1])
        t = (a2[e, 0] + a2[e, 1]) * nd
        acc = t if acc is None else acc + t
    logits = acc * (1.0 / 3.0) + bbar[None, :]
    logits_r[...] = logits
    mask = lax.broadcasted_iota(jnp.int32, logits.shape, 1) < NCLS
    xm = jnp.where(mask, logits, -1e30)
    mx = jnp.max(xm, axis=1, keepdims=True)
    ex = jnp.where(mask, jnp.exp(xm - mx), 0.0)
    probs_r[...] = ex / jnp.sum(ex, axis=1, keepdims=True)


def _row_spec(rank_prefix, r, minor):
    def imap(i):
        return tuple([0] * len(rank_prefix) + [i, 0])
    return pl.BlockSpec(tuple(rank_prefix) + (r, minor), imap)


def _full_spec(shape):
    return pl.BlockSpec(shape, lambda i: tuple(0 for _ in shape))


def kernel(x, edge_index_0, edge_index_1, edge_index_2,
           W0_0, b0_0, W0_1, b0_1, W0_2, b0_2,
           W1_0, b1_0, W1_1, b1_1, W1_2, b1_2,
           W2_0, b2_0, W2_1, b2_1, W2_2, b2_2):
    f32 = jnp.float32

    x_pad = jnp.pad(x, ((0, N_PAD - N), (0, 0)))
    pad_e = PAD_IDX + jnp.arange(E_PAD - E, dtype=jnp.int32) % (N_PAD - N)
    srcs, dsts = [], []
    for ei in (edge_index_0, edge_index_1, edge_index_2):
        srcs.append(jnp.concatenate([ei[0], pad_e]).reshape(NW, CPW, CHUNK))
        dsts.append(jnp.concatenate([ei[1], pad_e]).reshape(NW, CPW, CHUNK))
    src3 = jnp.stack(srcs)
    dst3 = jnp.stack(dsts)

    W0s = jnp.stack([W0_0, W0_1, W0_2])
    b0s = jnp.stack([b0_0, b0_1, b0_2])
    W1s = jnp.stack([W1_0, W1_1, W1_2])
    b1s = jnp.stack([b1_0, b1_1, b1_2])
    W2s = jnp.pad(jnp.stack([W2_0, W2_1, W2_2]), ((0, 0), (0, 0), (0, 88)))
    b2s = jnp.pad(jnp.stack([b2_0, b2_1, b2_2]), ((0, 0), (0, 88)))

    zrow128 = jnp.zeros((CHUNK, IN_F), f32)
    rowidx80 = jnp.arange(N_PAD // IN_F, dtype=jnp.int32)

    degf = _make_sc_degrees()(src3, dst3, zrow128, rowidx80)
    degp = jnp.sum(degf.reshape(6, NC, IN_F * IN_F)[:, :, :N_PAD],
                   axis=1)[:, :, None]

    r0 = 256
    grid0 = (N_PAD // r0,)
    xs = pl.pallas_call(
        _k0_body,
        grid=grid0,
        in_specs=[_row_spec((6,), r0, 1), _row_spec((), r0, IN_F)],
        out_specs=[_row_spec((), r0, IN_F)] * 3,
        out_shape=[jax.ShapeDtypeStruct((N_PAD, IN_F), f32)] * 3,
    )(degp, x_pad)

    agg0 = _make_sc_agg(3, IN_F, [0, 1, 2])(xs, src3, dst3, zrow128)

    r1 = 256
    ys = pl.pallas_call(
        _k1_body,
        grid=(N_PAD // r1,),
        in_specs=[
            _row_spec((6,), r1, 1),
            _row_spec((3, NC), r1, IN_F),
            _full_spec((3, IN_F, HID)),
            _full_spec((3, HID)),
        ],
        out_specs=[_row_spec((), r1, IN_F)] * 6,
        out_shape=[jax.ShapeDtypeStruct((N_PAD, IN_F), f32)] * 6,
    )(degp, agg0, W0s, b0s)

    agg1 = _make_sc_agg(6, IN_F, [0, 0, 1, 1, 2, 2])(ys, src3, dst3, zrow128)

    zs = pl.pallas_call(
        _k2_body,
        grid=(N_PAD // r1,),
        in_specs=[
            _row_spec((6,), r1, 1),
            _row_spec((6, NC), r1, IN_F),
            _full_spec((3, HID, HID)),
            _full_spec((3, HID)),
            _full_spec((3, HID, IN_F)),
        ],
        out_specs=[_row_spec((), r1, IN_F)] * 3,
        out_shape=[jax.ShapeDtypeStruct((N_PAD, IN_F), f32)] * 3,
    )(degp, agg1, W1s, b1s, W2s)

    agg2 = _make_sc_agg(3, IN_F, [0, 1, 2])(zs, src3, dst3, zrow128)

    logits_pad, probs_pad = pl.pallas_call(
        _k3_body,
        grid=(N_PAD // r1,),
        in_specs=[
            _row_spec((6,), r1, 1),
            _row_spec((3, NC), r1, IN_F),
            _full_spec((3, IN_F)),
        ],
        out_specs=[_row_spec((), r1, IN_F)] * 2,
        out_shape=[jax.ShapeDtypeStruct((N_PAD, IN_F), f32)] * 2,
    )(degp, agg2, b2s)

    return logits_pad[:N, :NCLS], probs_pad[:N, :NCLS]


# speedup vs baseline: 6.3812x; 1.0501x over previous
import functools

import jax
import jax.numpy as jnp
from jax import lax
from jax.experimental import pallas as pl
from jax.experimental.pallas import tpu as pltpu
from jax.experimental.pallas import tpu_sc as plsc

N = 10000
E = 100000
IN_F = 128
HID = 256
NCLS = 40
NET = 3

NC = 2
NS = 16
NW = NC * NS
CHUNK = 128
N_PAD = 10240
PAD_IDX = N
CPW = 25
E_PAD = NW * CPW * CHUNK
STRIPE = N_PAD // NS
WB = STRIPE // CHUNK


def _make_sc_agg(n_passes, feat, etype_of_pass):
    mesh = plsc.VectorSubcoreMesh(
        core_axis_name="c", subcore_axis_name="s", num_cores=NC,
        num_subcores=NS)

    def body(*refs):
        tabs = refs[:n_passes]
        src_r, dst_r, zrow_r = refs[n_passes:n_passes + 3]
        out_r = refs[n_passes + 3]
        rows0, rows1, sidx, didx, acc, sem0, sem1 = refs[n_passes + 4:]

        c = lax.axis_index("c")
        s = lax.axis_index("s")
        w = c * NS + s
        rb = s * STRIPE

        def load_idx(p):
            e = etype_of_pass[p]
            pltpu.sync_copy(src_r.at[e, w], sidx)
            pltpu.sync_copy(dst_r.at[e, w], didx)

        def zero_stripe():
            pltpu.sync_copy(zrow_r, rows1)
            for j in range(WB):
                pltpu.sync_copy(rows1, acc.at[pl.ds(rb + j * CHUNK, CHUNK)])

        load_idx(0)
        zero_stripe()
        plsc.subcore_barrier()
        pltpu.async_copy(tabs[0].at[sidx.at[0]], rows0, sem0)

        for p in range(n_passes):
            def step(i, carry):
                j = 2 * i
                pltpu.async_copy(tabs[p].at[sidx.at[j + 1]], rows1, sem1)
                pltpu.make_async_copy(
                    tabs[p].at[sidx.at[j]], rows0, sem0).wait()
                pltpu.sync_copy(rows0, acc.at[didx.at[j]], add=True)
                pltpu.async_copy(tabs[p].at[sidx.at[j + 2]], rows0, sem0)
                pltpu.make_async_copy(
                    tabs[p].at[sidx.at[j + 1]], rows1, sem1).wait()
                pltpu.sync_copy(rows1, acc.at[didx.at[j + 1]], add=True)
                return carry

            lax.fori_loop(0, (CPW - 1) // 2, step, 0)
            pltpu.make_async_copy(
                tabs[p].at[sidx.at[CPW - 1]], rows0, sem0).wait()
            pltpu.sync_copy(rows0, acc.at[didx.at[CPW - 1]], add=True)
            plsc.subcore_barrier()

            wb = pltpu.async_copy(acc.at[pl.ds(rb, STRIPE)],
                                  out_r.at[p, c, pl.ds(rb, STRIPE)], sem1)
            if p + 1 < n_passes:
                load_idx(p + 1)
                pltpu.async_copy(tabs[p + 1].at[sidx.at[0]], rows0, sem0)
                wb.wait()
                zero_stripe()
                plsc.subcore_barrier()
            else:
                wb.wait()

    tab_types = [jax.ShapeDtypeStruct((N_PAD, feat), jnp.float32)
                 for _ in range(n_passes)]

    def run(tables, src3, dst3, zrow):
        f = pl.kernel(
            body,
            out_type=jax.ShapeDtypeStruct((n_passes, NC, N_PAD, feat),
                                          jnp.float32),
            mesh=mesh,
            scratch_types=[
                pltpu.VMEM((CHUNK, feat), jnp.float32),
                pltpu.VMEM((CHUNK, feat), jnp.float32),
                pltpu.VMEM((CPW, CHUNK), jnp.int32),
                pltpu.VMEM((CPW, CHUNK), jnp.int32),
                pltpu.VMEM_SHARED((N_PAD, feat), jnp.float32),
                pltpu.SemaphoreType.DMA,
                pltpu.SemaphoreType.DMA,
            ],
        )
        return f(*tables, src3, dst3, zrow)

    del tab_types
    return run


def _make_sc_degrees():
    mesh = plsc.VectorSubcoreMesh(
        core_axis_name="c", subcore_axis_name="s", num_cores=NC,
        num_subcores=NS)
    DR = N_PAD // IN_F

    def body(src_r, dst_r, zrow_r, rowidx_r, out_r,
             deg2d, didx, rowidx, acc, sem0):
        del sem0
        c = lax.axis_index("c")
        s = lax.axis_index("s")
        w = c * NS + s

        pltpu.sync_copy(rowidx_r, rowidx)
        ones16 = jnp.ones((16,), jnp.float32)

        for p in range(6):
            e = p // 2
            idx_r = src_r if p % 2 == 0 else dst_r
            pltpu.sync_copy(zrow_r.at[pl.ds(0, 8)], acc.at[pl.ds(s * 8, 8)])
            pltpu.sync_copy(zrow_r.at[pl.ds(0, DR)], deg2d)
            pltpu.sync_copy(idx_r.at[e, w], didx)

            def step(g, carry):
                r = g // 8
                col = (g % 8) * 16
                v = didx[r, pl.ds(col, 16)]
                plsc.addupdate_scatter(
                    deg2d,
                    [lax.shift_right_logical(v, 7),
                     lax.bitwise_and(v, 127)],
                    ones16)
                return carry

            lax.fori_loop(0, CPW * 8, step, 0)
            plsc.subcore_barrier()
            pltpu.sync_copy(deg2d, acc.at[rowidx], add=True)
            plsc.subcore_barrier()
            pltpu.sync_copy(acc.at[pl.ds(s * 8, 8)],
                            out_r.at[p, c, pl.ds(s * 8, 8)])
            plsc.subcore_barrier()

    def run(src3, dst3, zrow, rowidx):
        f = pl.kernel(
            body,
            out_type=jax.ShapeDtypeStruct((6, NC, IN_F, IN_F), jnp.float32),
            mesh=mesh,
            compiler_params=pltpu.CompilerParams(needs_layout_passes=False),
            scratch_types=[
                pltpu.VMEM((DR, IN_F), jnp.float32),
                pltpu.VMEM((CPW, CHUNK), jnp.int32),
                pltpu.VMEM((DR,), jnp.int32),
                pltpu.VMEM_SHARED((IN_F, IN_F), jnp.float32),
                pltpu.SemaphoreType.DMA,
            ],
        )
        return f(src3, dst3, zrow, rowidx)

    return run


def _norm_col(deg):
    return jnp.where(deg > 0.0, lax.rsqrt(jnp.maximum(deg, 1.0)), 0.0)


def _k0_body(degp_r, x_r, xs0_r, xs1_r, xs2_r):
    degp = degp_r[...]
    x = x_r[...]
    outs = (xs0_r, xs1_r, xs2_r)
    for e in range(NET):
        ns = _norm_col(degp[2 * e])
        outs[e][...] = x * ns


def _k1_body(degp_r, a0_r, w_r, b_r, *y_refs):
    degp = degp_r[...]
    a0 = a0_r[...]
    w = w_r[...]
    b = b_r[...]
    bbar = (b[0] + b[1] + b[2]) * (1.0 / 3.0)
    acc = None
    for e in range(NET):
        nd = _norm_col(degp[2 * e + 1])
        part = (a0[e, 0] + a0[e, 1]) * nd
        t = jnp.dot(part, w[e], preferred_element_type=jnp.float32)
        acc = t if acc is None else acc + t
    h1 = jnp.maximum(acc * (1.0 / 3.0) + bbar[None, :], 0.0)
    for e in range(NET):
        ns = _norm_col(degp[2 * e])
        t = h1 * ns
        y_refs[2 * e][...] = t[:, :IN_F]
        y_refs[2 * e + 1][...] = t[:, IN_F:]


def _k2_body(degp_r, a1_r, w1_r, b1_r, w2_r, z0_r, z1_r, z2_r):
    degp = degp_r[...]
    a1 = a1_r[...]
    w1 = w1_r[...]
    b1 = b1_r[...]
    w2 = w2_r[...]
    bbar = (b1[0] + b1[1] + b1[2]) * (1.0 / 3.0)
    acc = None
    for e in range(NET):
        nd = _norm_col(degp[2 * e + 1])
        t = None
        for h in range(2):
            part = a1[2 * e + h, 0] + a1[2 * e + h, 1]
            m = jnp.dot(part, w1[e, h * IN_F:(h + 1) * IN_F, :],
                        preferred_element_type=jnp.float32)
            t = m if t is None else t + m
        t = t * nd
        acc = t if acc is None else acc + t
    h2 = jnp.maximum(acc * (1.0 / 3.0) + bbar[None, :], 0.0)
    z_refs = (z0_r, z1_r, z2_r)
    for e in range(NET):
        ns = _norm_col(degp[2 * e])
        z_refs[e][...] = jnp.dot(h2 * ns, w2[e],
                                 preferred_element_type=jnp.float32)


def _k3_body(degp_r, a2_r, b2_r, logits_r, probs_r):
    degp = degp_r[...]
    a2 = a2_r[...]
    b2 = b2_r[...]
    bbar = (b2[0] + b2[1] + b2[2]) * (1.0 / 3.0)
    acc = None
    for e in range(NET):
        nd = _norm_col(degp[2 * e + 1])
        t = (a2[e, 0] + a2[e, 1]) * nd
        acc = t if acc is None else acc + t
    logits = acc * (1.0 / 3.0) + bbar[None, :]
    logits_r[...] = logits
    mask = lax.broadcasted_iota(jnp.int32, logits.shape, 1) < NCLS
    xm = jnp.where(mask, logits, -1e30)
    mx = jnp.max(xm, axis=1, keepdims=True)
    ex = jnp.where(mask, jnp.exp(xm - mx), 0.0)
    probs_r[...] = ex / jnp.sum(ex, axis=1, keepdims=True)


def _row_spec(rank_prefix, r, minor):
    def imap(i):
        return tuple([0] * len(rank_prefix) + [i, 0])
    return pl.BlockSpec(tuple(rank_prefix) + (r, minor), imap)


def _full_spec(shape):
    return pl.BlockSpec(shape, lambda i: tuple(0 for _ in shape))


def kernel(x, edge_index_0, edge_index_1, edge_index_2,
           W0_0, b0_0, W0_1, b0_1, W0_2, b0_2,
           W1_0, b1_0, W1_1, b1_1, W1_2, b1_2,
           W2_0, b2_0, W2_1, b2_1, W2_2, b2_2):
    f32 = jnp.float32

    x_pad = jnp.pad(x, ((0, N_PAD - N), (0, 0)))
    pad_e = PAD_IDX + jnp.arange(E_PAD - E, dtype=jnp.int32) % (N_PAD - N)
    srcs, dsts = [], []
    for ei in (edge_index_0, edge_index_1, edge_index_2):
        srcs.append(jnp.concatenate([ei[0], pad_e]).reshape(NW, CPW, CHUNK))
        dsts.append(jnp.concatenate([ei[1], pad_e]).reshape(NW, CPW, CHUNK))
    src3 = jnp.stack(srcs)
    dst3 = jnp.stack(dsts)

    W0s = jnp.stack([W0_0, W0_1, W0_2])
    b0s = jnp.stack([b0_0, b0_1, b0_2])
    W1s = jnp.stack([W1_0, W1_1, W1_2])
    b1s = jnp.stack([b1_0, b1_1, b1_2])
    W2s = jnp.pad(jnp.stack([W2_0, W2_1, W2_2]), ((0, 0), (0, 0), (0, 88)))
    b2s = jnp.pad(jnp.stack([b2_0, b2_1, b2_2]), ((0, 0), (0, 88)))

    zrow128 = jnp.zeros((CHUNK, IN_F), f32)
    rowidx80 = jnp.arange(N_PAD // IN_F, dtype=jnp.int32)

    degf = _make_sc_degrees()(src3, dst3, zrow128, rowidx80)
    degp = jnp.sum(degf.reshape(6, NC, IN_F * IN_F)[:, :, :N_PAD],
                   axis=1)[:, :, None]

    r0 = 256
    grid0 = (N_PAD // r0,)
    xs = pl.pallas_call(
        _k0_body,
        grid=grid0,
        in_specs=[_row_spec((6,), r0, 1), _row_spec((), r0, IN_F)],
        out_specs=[_row_spec((), r0, IN_F)] * 3,
        out_shape=[jax.ShapeDtypeStruct((N_PAD, IN_F), f32)] * 3,
    )(degp, x_pad)

    agg0 = _make_sc_agg(3, IN_F, [0, 1, 2])(xs, src3, dst3, zrow128)

    r1 = 256
    ys = pl.pallas_call(
        _k1_body,
        grid=(N_PAD // r1,),
        in_specs=[
            _row_spec((6,), r1, 1),
            _row_spec((3, NC), r1, IN_F),
            _full_spec((3, IN_F, HID)),
            _full_spec((3, HID)),
        ],
        out_specs=[_row_spec((), r1, IN_F)] * 6,
        out_shape=[jax.ShapeDtypeStruct((N_PAD, IN_F), f32)] * 6,
    )(degp, agg0, W0s, b0s)

    agg1 = _make_sc_agg(6, IN_F, [0, 0, 1, 1, 2, 2])(ys, src3, dst3, zrow128)

    zs = pl.pallas_call(
        _k2_body,
        grid=(N_PAD // r1,),
        in_specs=[
            _row_spec((6,), r1, 1),
            _row_spec((6, NC), r1, IN_F),
            _full_spec((3, HID, HID)),
            _full_spec((3, HID)),
            _full_spec((3, HID, IN_F)),
        ],
        out_specs=[_row_spec((), r1, IN_F)] * 3,
        out_shape=[jax.ShapeDtypeStruct((N_PAD, IN_F), f32)] * 3,
    )(degp, agg1, W1s, b1s, W2s)

    agg2 = _make_sc_agg(3, IN_F, [0, 1, 2])(zs, src3, dst3, zrow128)

    logits_pad, probs_pad = pl.pallas_call(
        _k3_body,
        grid=(N_PAD // r1,),
        in_specs=[
            _row_spec((6,), r1, 1),
            _row_spec((3, NC), r1, IN_F),
            _full_spec((3, IN_F)),
        ],
        out_specs=[_row_spec((), r1, IN_F)] * 2,
        out_shape=[jax.ShapeDtypeStruct((N_PAD, IN_F), f32)] * 2,
    )(degp, agg2, b2s)

    return logits_pad[:N, :NCLS], probs_pad[:N, :NCLS]
